# baseline (speedup 1.0000x reference)
import functools

import jax
import jax.numpy as jnp
from jax import lax
from jax.experimental import pallas as pl
from jax.experimental.pallas import tpu as pltpu
from jax.experimental.pallas import tpu_sc as plsc

A = 20000
A_PAD = 20480
NB = 10
SB = 16
N = 8
M = 20
TOTAL = 256
MAX_FG = 128
POS_OV = 0.7
NEG_OV = 0.3
IMG = 600.0


def _tc_label_body(anch_ref, coeff_ref, gt_ref, enc_ref,
                   ov_scr, ins_scr, acc_scr, cmax_smem):
    b = pl.program_id(0)
    p = pl.program_id(1)
    i = pl.program_id(2)

    @pl.when(p == 0)
    def _phase0():
        ax1 = anch_ref[0]
        ay1 = anch_ref[1]
        ax2 = anch_ref[2]
        ay2 = anch_ref[3]
        w = ax2 - ax1 + 1.0
        h = ay2 - ay1 + 1.0
        cx = ax1 + 0.5 * w
        cy = ay1 + 0.5 * h
        dx = coeff_ref[0, 0]
        dy = coeff_ref[0, 1]
        dw = coeff_ref[0, 2]
        dh = coeff_ref[0, 3]
        pcx = dx * w + cx
        pcy = dy * h + cy
        pw = jnp.exp(dw) * w
        ph = jnp.exp(dh) * h
        px1 = pcx - 0.5 * pw
        py1 = pcy - 0.5 * ph
        px2 = pcx + 0.5 * pw - 1.0
        py2 = pcy + 0.5 * ph - 1.0
        inside = (px1 >= 0.0) & (py1 >= 0.0) & (px2 < IMG) & (py2 < IMG)
        ins_scr[i] = inside.astype(jnp.int32)
        area_a = (px2 - px1 + 1.0) * (py2 - py1 + 1.0)
        for m in range(M):
            gx1 = gt_ref[0, 0, m, 0]
            gy1 = gt_ref[0, 0, m, 1]
            gx2 = gt_ref[0, 0, m, 2]
            gy2 = gt_ref[0, 0, m, 3]
            iw = jnp.maximum(jnp.minimum(px2, gx2) - jnp.maximum(px1, gx1) + 1.0, 0.0)
            ih = jnp.maximum(jnp.minimum(py2, gy2) - jnp.maximum(py1, gy1) + 1.0, 0.0)
            inter = iw * ih
            area_g = (gx2 - gx1 + 1.0) * (gy2 - gy1 + 1.0)
            ov = inter / (area_a + area_g - inter)
            ov = jnp.where(inside, ov, -1.0)
            ov_scr[i, m] = ov

            @pl.when(i == 0)
            def _():
                acc_scr[m] = ov

            @pl.when(i > 0)
            def _():
                acc_scr[m] = jnp.maximum(acc_scr[m], ov)

        @pl.when(i == NB - 1)
        def _():
            for m in range(M):
                cmax_smem[b, m] = jnp.max(acc_scr[m])

    @pl.when(p == 1)
    def _phase1():
        ov0 = ov_scr[i, 0]
        cm0 = cmax_smem[b, 0]
        abox = ov0 == cm0
        bbox = ov0 >= POS_OV
        anyneg = ov0 >= NEG_OV
        best = ov0
        bidx = jnp.zeros((SB, 128), jnp.int32)
        for m in range(1, M):
            ovm = ov_scr[i, m]
            cmm = cmax_smem[b, m]
            abox = abox | (ovm == cmm)
            bbox = bbox | (ovm >= POS_OV)
            anyneg = anyneg | (ovm >= NEG_OV)
            upd = ovm > best
            bidx = jnp.where(upd, m, bidx)
            best = jnp.maximum(best, ovm)
        ins = ins_scr[i] != 0
        pos = abox | bbox
        fg = pos & ins
        bg = (~anyneg) & (~pos) & ins
        lab = fg.astype(jnp.int32) * 2 + bg.astype(jnp.int32)
        enc_ref[0] = lab + bidx * 4


def _tc_label_call(anch_r, coeff_r, gt_r):
    return pl.pallas_call(
        _tc_label_body,
        grid=(N, 2, NB),
        in_specs=[
            pl.BlockSpec((4, SB, 128), lambda b, p, i: (0, i, 0)),
            pl.BlockSpec((1, 4, SB, 128), lambda b, p, i: (b, 0, i, 0)),
            pl.BlockSpec((1, 1, M, 4), lambda b, p, i: (b, 0, 0, 0)),
        ],
        out_specs=pl.BlockSpec((1, SB, 128), lambda b, p, i: (b, i, 0)),
        out_shape=jax.ShapeDtypeStruct((N, A_PAD // 128, 128), jnp.int32),
        scratch_shapes=[
            pltpu.VMEM((NB, M, SB, 128), jnp.float32),
            pltpu.VMEM((NB, SB, 128), jnp.int32),
            pltpu.VMEM((M, SB, 128), jnp.float32),
            pltpu.SMEM((N, M), jnp.float32),
        ],
    )(anch_r, coeff_r, gt_r)


def _b2i(x):
    return jnp.where(x, jnp.int32(1), jnp.int32(0))


def _sc_select_body(enc_hbm, anch_hbm, gt_hbm,
                    sel_out, fgm_out, tpre_out,
                    enc_v, ax1_v, ay1_v, ax2_v, ay2_v,
                    g1_v, g2_v, g3_v, g4_v,
                    sel_v, fgm_v, dx_v, dy_v, rw_v, rh_v, sem):
    c = lax.axis_index("c")
    s = lax.axis_index("s")
    wid = s * 2 + c

    @pl.when(wid < N)
    def _work():
        b = wid
        cps = []
        for k, ref in enumerate((ax1_v, ay1_v, ax2_v, ay2_v)):
            cps.append(pltpu.async_copy(anch_hbm.at[k], ref, sem))
        for k, ref in enumerate((g1_v, g2_v, g3_v, g4_v)):
            cps.append(pltpu.async_copy(gt_hbm.at[b, k], ref, sem))
        pltpu.sync_copy(enc_hbm.at[b], enc_v)

        zeros16 = jnp.zeros((16,), jnp.int32)
        for j in range(TOTAL // 16):
            sel_v[pl.ds(j * 16, 16)] = zeros16

        def count_body(i, acc):
            v = enc_v[pl.ds(i * 16, 16)]
            return acc + plsc.all_reduce_population_count((v & 3) == 2)

        total_fg = lax.fori_loop(0, A_PAD // 16, count_body, zeros16)
        max_bg = TOTAL - jnp.minimum(total_fg, MAX_FG)

        iota16 = lax.iota(jnp.int32, 16)

        def scan_body(i, carry):
            cfg, cbg, ck, idxv = carry
            v = enc_v[pl.ds(i * 16, 16)]
            labv = v & 3
            fg = labv == 2
            bg = labv == 1
            fcs = plsc.cumsum(_b2i(fg))
            bcs = plsc.cumsum(_b2i(bg))
            keep = (fg & ((cfg + fcs) <= MAX_FG)) | (bg & ((cbg + bcs) <= max_bg))
            kcs = plsc.cumsum(_b2i(keep))
            pos = ck + kcs - 1
            plsc.store_scatter(sel_v, [pos], idxv, mask=keep)
            return (cfg + plsc.all_reduce_population_count(fg),
                    cbg + plsc.all_reduce_population_count(bg),
                    ck + plsc.all_reduce_population_count(keep),
                    idxv + 16)

        lax.fori_loop(0, A_PAD // 16, scan_body,
                      (zeros16, zeros16, zeros16, iota16))

        for cp in cps:
            cp.wait()

        for j in range(TOTAL // 16):
            sl = pl.ds(j * 16, 16)
            sidx = sel_v[sl]
            e = plsc.load_gather(enc_v, [sidx])
            fgm_v[sl] = _b2i((e & 3) == 2)
            am = e >> 2
            x1 = plsc.load_gather(ax1_v, [sidx])
            y1 = plsc.load_gather(ay1_v, [sidx])
            x2 = plsc.load_gather(ax2_v, [sidx])
            y2 = plsc.load_gather(ay2_v, [sidx])
            gx1 = plsc.load_gather(g1_v, [am])
            gy1 = plsc.load_gather(g2_v, [am])
            gx2 = plsc.load_gather(g3_v, [am])
            gy2 = plsc.load_gather(g4_v, [am])
            aw = x2 - x1 + 1.0
            ah = y2 - y1 + 1.0
            acx = x1 + 0.5 * aw
            acy = y1 + 0.5 * ah
            gw = gx2 - gx1 + 1.0
            gh = gy2 - gy1 + 1.0
            gcx = gx1 + 0.5 * gw
            gcy = gy1 + 0.5 * gh
            dx_v[sl] = (gcx - acx) / aw
            dy_v[sl] = (gcy - acy) / ah
            rw_v[sl] = gw / aw
            rh_v[sl] = gh / ah

        pltpu.sync_copy(sel_v, sel_out.at[b])
        pltpu.sync_copy(fgm_v, fgm_out.at[b])
        pltpu.sync_copy(dx_v, tpre_out.at[b, 0])
        pltpu.sync_copy(dy_v, tpre_out.at[b, 1])
        pltpu.sync_copy(rw_v, tpre_out.at[b, 2])
        pltpu.sync_copy(rh_v, tpre_out.at[b, 3])


def _sc_select_call(enc, anch4, gt4):
    f = pl.kernel(
        _sc_select_body,
        out_type=[
            jax.ShapeDtypeStruct((N, TOTAL), jnp.int32),
            jax.ShapeDtypeStruct((N, TOTAL), jnp.int32),
            jax.ShapeDtypeStruct((N, 4, TOTAL), jnp.float32),
        ],
        mesh=plsc.VectorSubcoreMesh(core_axis_name="c", subcore_axis_name="s",
                                    num_cores=2, num_subcores=16),
        compiler_params=pltpu.CompilerParams(needs_layout_passes=False),
        scratch_types=[
            pltpu.VMEM((A_PAD,), jnp.int32),
            pltpu.VMEM((A_PAD,), jnp.float32),
            pltpu.VMEM((A_PAD,), jnp.float32),
            pltpu.VMEM((A_PAD,), jnp.float32),
            pltpu.VMEM((A_PAD,), jnp.float32),
            pltpu.VMEM((24,), jnp.float32),
            pltpu.VMEM((24,), jnp.float32),
            pltpu.VMEM((24,), jnp.float32),
            pltpu.VMEM((24,), jnp.float32),
            pltpu.VMEM((TOTAL,), jnp.int32),
            pltpu.VMEM((TOTAL,), jnp.int32),
            pltpu.VMEM((TOTAL,), jnp.float32),
            pltpu.VMEM((TOTAL,), jnp.float32),
            pltpu.VMEM((TOTAL,), jnp.float32),
            pltpu.VMEM((TOTAL,), jnp.float32),
            pltpu.SemaphoreType.DMA,
        ],
    )
    return f(enc, anch4, gt4)


def _tc_log_body(x_ref, o_ref):
    r = lax.broadcasted_iota(jnp.int32, (4 * N, TOTAL), 0)
    mask = (r % 4) >= 2
    x = x_ref[...]
    o_ref[...] = jnp.where(mask, jnp.log(x), x)


def _tc_log_call(x):
    return pl.pallas_call(
        _tc_log_body,
        out_shape=jax.ShapeDtypeStruct((4 * N, TOTAL), jnp.float32),
    )(x)


def kernel(anchors, gt_boxes, bbox_coeff):
    anchors = anchors.astype(jnp.float32)
    gt_boxes = gt_boxes.astype(jnp.float32)
    bbox_coeff = bbox_coeff.astype(jnp.float32)

    anchors_p = jnp.pad(anchors, ((0, A_PAD - A), (0, 0)), constant_values=-1e6)
    coeff_p = jnp.pad(bbox_coeff, ((0, 0), (0, A_PAD - A), (0, 0)))

    anch_t = anchors_p.T
    anch_r = anch_t.reshape(4, A_PAD // 128, 128)
    coeff_r = coeff_p.transpose(0, 2, 1).reshape(N, 4, A_PAD // 128, 128)
    gt_r = gt_boxes.reshape(N, 1, M, 4)

    enc = _tc_label_call(anch_r, coeff_r, gt_r)

    gt4 = jnp.pad(gt_boxes.transpose(0, 2, 1), ((0, 0), (0, 0), (0, 4)),
                  constant_values=1.0)

    sel, fgm, tpre = _sc_select_call(enc.reshape(N, A_PAD), anch_t, gt4)

    tlog = _tc_log_call(tpre.reshape(4 * N, TOTAL))
    target = tlog.reshape(N, 4, TOTAL).transpose(0, 2, 1)

    return sel, fgm.astype(bool), target
